# Initial kernel scaffold; baseline (speedup 1.0000x reference)
import jax
import jax.numpy as jnp
from jax.experimental import pallas as pl


def kernel(x, cond, W_ef, b_ef, Wg_ef, Wb_ef, W_ew, b_ew, W_ep, b_ep, W_nf, b_nf, Wg_nf, Wb_nf, edge_index):
    raise NotImplementedError("write your pallas kernel here")


# speedup vs baseline: 1.9879x; 1.9879x over previous
import functools

import jax
import jax.numpy as jnp
from jax import lax
from jax.experimental import pallas as pl
from jax.experimental.pallas import tpu as pltpu
from jax.experimental.pallas import tpu_sc as plsc

N = 10000
E = 320000
GRP = 128
NGRP = E // GRP
NC, NS = 2, 16
NW = NC * NS
NZG = N // GRP
NZT = N - NZG * GRP

f32 = jnp.float32
i32 = jnp.int32


def _node_body(x_ref, cond_ref, wgef_ref, wbef_ref, wnf_ref, bnf_ref,
               wgnf_ref, wbnf_ref, gb_ref, nf_ref):
    x = x_ref[...]
    cond = cond_ref[...]
    g = jnp.dot(cond, wgef_ref[...], preferred_element_type=f32)
    b = jnp.dot(cond, wbef_ref[...], preferred_element_type=f32)
    gb_ref[...] = jnp.concatenate([g, b], axis=1)
    hn = jnp.dot(x, wnf_ref[...], preferred_element_type=f32) + bnf_ref[...]
    nf_ref[...] = (jnp.dot(cond, wgnf_ref[...], preferred_element_type=f32) * hn
                   + jnp.dot(cond, wbnf_ref[...], preferred_element_type=f32))


def _node_precompute(x, cond, Wg_ef, Wb_ef, W_nf, b_nf, Wg_nf, Wb_nf):
    return pl.pallas_call(
        _node_body,
        out_shape=(jax.ShapeDtypeStruct((N, 128), f32),
                   jax.ShapeDtypeStruct((N, 128), f32)),
    )(x, cond, Wg_ef, Wb_ef, W_nf, b_nf.reshape(1, 128), Wg_nf, Wb_nf)


def _sc_gather_body(src_hbm, dst_hbm, x_hbm, gb_hbm, p_out, gbe_out,
                    src_v, dst_v, xs_v, xd_v, sem):
    wid = lax.axis_index("s") * NC + lax.axis_index("c")
    n_j = (NGRP - wid + NW - 1) // NW

    def grp(j, carry):
        g = wid + NW * j
        base = g * GRP
        pltpu.sync_copy(src_hbm.at[pl.ds(base, GRP)], src_v)
        pltpu.sync_copy(dst_hbm.at[pl.ds(base, GRP)], dst_v)
        pltpu.async_copy(x_hbm.at[src_v], xs_v, sem).wait()
        pltpu.async_copy(x_hbm.at[dst_v], xd_v, sem).wait()

        def mulrow(i, c):
            for k in range(8):
                s = pl.ds(k * 16, 16)
                xs_v[i, s] = xs_v[i, s] * xd_v[i, s]
            return c
        lax.fori_loop(0, GRP, mulrow, 0)
        pltpu.sync_copy(xs_v, p_out.at[pl.ds(base, GRP), :])
        pltpu.async_copy(gb_hbm.at[dst_v], xd_v, sem).wait()
        pltpu.sync_copy(xd_v, gbe_out.at[pl.ds(base, GRP), :])
        return carry
    lax.fori_loop(0, n_j, grp, 0)


def _sc_gather(src, dst, x, gb):
    mesh = plsc.VectorSubcoreMesh(core_axis_name="c", subcore_axis_name="s")
    k = functools.partial(
        pl.kernel,
        out_type=(jax.ShapeDtypeStruct((E, 128), f32),
                  jax.ShapeDtypeStruct((E, 128), f32)),
        mesh=mesh,
        scratch_types=[
            pltpu.VMEM((GRP,), i32),
            pltpu.VMEM((GRP,), i32),
            pltpu.VMEM((GRP, 128), f32),
            pltpu.VMEM((GRP, 128), f32),
            pltpu.SemaphoreType.DMA,
        ],
    )(_sc_gather_body)
    return k(src, dst, x, gb)


BE = 2000


def _edge1_body(p_ref, gb_ref, wef_ref, bef_ref, wew_ref, bew_ref,
                ef_ref, ex_ref):
    p = p_ref[...]
    gb = gb_ref[...]
    h = jnp.dot(p, wef_ref[...], preferred_element_type=f32) + bef_ref[...]
    ef = gb[:, :64] * h + gb[:, 64:]
    ef_ref[...] = ef
    logits = jnp.dot(ef, wew_ref[...], preferred_element_type=f32) + bew_ref[...]
    ex_ref[...] = jnp.exp(logits)


def _edge_mlp1(p, gbe, W_ef, b_ef, W_ew, b_ew):
    nblk = E // BE
    return pl.pallas_call(
        _edge1_body,
        grid=(nblk,),
        in_specs=[
            pl.BlockSpec((BE, 128), lambda i: (i, 0)),
            pl.BlockSpec((BE, 128), lambda i: (i, 0)),
            pl.BlockSpec((128, 64), lambda i: (0, 0)),
            pl.BlockSpec((1, 64), lambda i: (0, 0)),
            pl.BlockSpec((64, 8), lambda i: (0, 0)),
            pl.BlockSpec((1, 8), lambda i: (0, 0)),
        ],
        out_specs=[
            pl.BlockSpec((BE, 64), lambda i: (i, 0)),
            pl.BlockSpec((BE, 8), lambda i: (i, 0)),
        ],
        out_shape=(jax.ShapeDtypeStruct((E, 64), f32),
                   jax.ShapeDtypeStruct((E, 8), f32)),
    )(p, gbe, W_ef, b_ef.reshape(1, 64), W_ew, b_ew.reshape(1, 8))


FLAT = E * 8
NFLAT = N * 8
FGRP = GRP * 8


def _sc_seg8_body(fidx_hbm, exf_hbm, part_out, fidx_v, exf_v, acc_v, sem):
    wid = lax.axis_index("s") * NC + lax.axis_index("c")

    def zrow(i, c):
        acc_v[pl.ds(i * 16, 16)] = jnp.zeros((16,), f32)
        return c
    lax.fori_loop(0, NFLAT // 16, zrow, 0)

    n_j = (NGRP - wid + NW - 1) // NW

    def grp(j, carry):
        g = wid + NW * j
        base = g * FGRP
        pltpu.sync_copy(fidx_hbm.at[pl.ds(base, FGRP)], fidx_v)
        pltpu.sync_copy(exf_hbm.at[pl.ds(base, FGRP)], exf_v)
        for j2 in range(FGRP // 16):
            sl = pl.ds(j2 * 16, 16)
            plsc.addupdate_scatter(acc_v, [fidx_v[sl]], exf_v[sl])
        return carry
    lax.fori_loop(0, n_j, grp, 0)
    pltpu.sync_copy(acc_v, part_out.at[wid])


def _sc_seg8(fidx, exf):
    mesh = plsc.VectorSubcoreMesh(core_axis_name="c", subcore_axis_name="s")
    k = functools.partial(
        pl.kernel,
        out_type=jax.ShapeDtypeStruct((NW, NFLAT), f32),
        mesh=mesh,
        scratch_types=[
            pltpu.VMEM((FGRP,), i32),
            pltpu.VMEM((FGRP,), f32),
            pltpu.VMEM((NFLAT,), f32),
            pltpu.SemaphoreType.DMA,
        ],
        compiler_params=pltpu.CompilerParams(needs_layout_passes=False),
    )(_sc_seg8_body)
    return k(fidx, exf)


def _reduce_body(part_ref, den_ref):
    @pl.when(pl.program_id(0) == 0)
    def _init():
        den_ref[...] = jnp.zeros_like(den_ref)
    den_ref[...] += part_ref[0]


def _reduce_partials(part):
    return pl.pallas_call(
        _reduce_body,
        grid=(NW,),
        in_specs=[pl.BlockSpec((1, N, 8), lambda i: (i, 0, 0))],
        out_specs=pl.BlockSpec((N, 8), lambda i: (0, 0)),
        out_shape=jax.ShapeDtypeStruct((N, 8), f32),
    )(part.reshape(NW, N, 8))


def _sc_dd_body(dst_hbm, den_hbm, dd_out, dst_v, ridx_v, dd_v, sem):
    wid = lax.axis_index("s") * NC + lax.axis_index("c")
    n_j = (NGRP - wid + NW - 1) // NW

    def grp(j, carry):
        g = wid + NW * j
        base = g * GRP
        pltpu.sync_copy(dst_hbm.at[pl.ds(base, GRP)], dst_v)

        def shiftrow(i, c):
            sl = pl.ds(i * 16, 16)
            ridx_v[sl] = lax.shift_right_logical(dst_v[sl], 4)
            return c
        lax.fori_loop(0, GRP // 16, shiftrow, 0)
        pltpu.async_copy(den_hbm.at[ridx_v], dd_v, sem).wait()
        pltpu.sync_copy(dd_v, dd_out.at[pl.ds(base, GRP), :])
        return carry
    lax.fori_loop(0, n_j, grp, 0)


def _sc_dd(dst, den):
    mesh = plsc.VectorSubcoreMesh(core_axis_name="c", subcore_axis_name="s")
    k = functools.partial(
        pl.kernel,
        out_type=jax.ShapeDtypeStruct((E, 128), f32),
        mesh=mesh,
        scratch_types=[
            pltpu.VMEM((GRP,), i32),
            pltpu.VMEM((GRP,), i32),
            pltpu.VMEM((GRP, 128), f32),
            pltpu.SemaphoreType.DMA,
        ],
    )(_sc_dd_body)
    return k(dst, den.reshape(N // 16, 128))


def _edge2_body(ef_ref, ex_ref, ddr_ref, dst_ref, wep_ref, bep_ref, ew_ref):
    ddr = ddr_ref[...]
    mod = lax.bitwise_and(dst_ref[...], 15)
    dd = ddr[:, 0:8]
    for m in range(1, 16):
        dd = jnp.where(mod == m, ddr[:, 8 * m:8 * m + 8], dd)
    q = ex_ref[...] / (dd + 1e-9)
    s = jnp.mean(q, axis=1, keepdims=True)
    ef2 = ef_ref[...] * s
    ep = jnp.dot(ef2, wep_ref[...], preferred_element_type=f32) + bep_ref[...]
    ew_ref[...] = s * ep


def _edge_mlp2(ef, ex, ddr, dst, W_ep, b_ep):
    nblk = E // BE
    return pl.pallas_call(
        _edge2_body,
        grid=(nblk,),
        in_specs=[
            pl.BlockSpec((BE, 64), lambda i: (i, 0)),
            pl.BlockSpec((BE, 8), lambda i: (i, 0)),
            pl.BlockSpec((BE, 128), lambda i: (i, 0)),
            pl.BlockSpec((BE, 1), lambda i: (i, 0)),
            pl.BlockSpec((64, 128), lambda i: (0, 0)),
            pl.BlockSpec((1, 128), lambda i: (0, 0)),
        ],
        out_specs=pl.BlockSpec((BE, 128), lambda i: (i, 0)),
        out_shape=jax.ShapeDtypeStruct((E, 128), f32),
    )(ef, ex, ddr, dst.reshape(E, 1), W_ep, b_ep.reshape(1, 128))


def _sc_scatter_body(src_hbm, dst_hbm, ew_hbm, nf_hbm, part_out,
                     src_v, dst_v, ew_v, nf_v, out_sp, sem):
    c = lax.axis_index("c")
    s = lax.axis_index("s")

    def zrow(i, cc):
        for k in range(8):
            ew_v[i, pl.ds(k * 16, 16)] = jnp.zeros((16,), f32)
        return cc
    lax.fori_loop(0, GRP, zrow, 0)

    def zcopy(j, cc):
        r = (s + NS * j) * GRP
        pltpu.sync_copy(ew_v, out_sp.at[pl.ds(r, GRP), :])
        return cc
    lax.fori_loop(0, (NZG - s + NS - 1) // NS, zcopy, 0)

    @pl.when(s == NS - 1)
    def _ztail():
        pltpu.sync_copy(ew_v.at[pl.ds(0, NZT), :],
                        out_sp.at[pl.ds(NZG * GRP, NZT), :])
    plsc.subcore_barrier()

    half = NGRP // NC
    n_j = (half - s + NS - 1) // NS

    def grp(j, carry):
        g = c * half + s + NS * j
        base = g * GRP
        pltpu.sync_copy(src_hbm.at[pl.ds(base, GRP)], src_v)
        pltpu.sync_copy(dst_hbm.at[pl.ds(base, GRP)], dst_v)
        pltpu.async_copy(nf_hbm.at[src_v], nf_v, sem).wait()
        pltpu.sync_copy(ew_hbm.at[pl.ds(base, GRP), :], ew_v)

        def mulrow(i, cc):
            for k in range(8):
                sl = pl.ds(k * 16, 16)
                ew_v[i, sl] = ew_v[i, sl] * nf_v[i, sl]
            return cc
        lax.fori_loop(0, GRP, mulrow, 0)
        pltpu.sync_copy(ew_v, out_sp.at[dst_v], add=True)
        return carry
    lax.fori_loop(0, n_j, grp, 0)
    plsc.subcore_barrier()

    def wcopy(j, cc):
        r = (s + NS * j) * GRP
        pltpu.sync_copy(out_sp.at[pl.ds(r, GRP), :], ew_v)
        pltpu.sync_copy(ew_v, part_out.at[c, pl.ds(r, GRP), :])
        return cc
    lax.fori_loop(0, (NZG - s + NS - 1) // NS, wcopy, 0)

    @pl.when(s == NS - 1)
    def _wtail():
        pltpu.sync_copy(out_sp.at[pl.ds(NZG * GRP, NZT), :],
                        ew_v.at[pl.ds(0, NZT), :])
        pltpu.sync_copy(ew_v.at[pl.ds(0, NZT), :],
                        part_out.at[c, pl.ds(NZG * GRP, NZT), :])


def _sc_scatter(src, dst, ew, nf):
    mesh = plsc.VectorSubcoreMesh(core_axis_name="c", subcore_axis_name="s")
    k = functools.partial(
        pl.kernel,
        out_type=jax.ShapeDtypeStruct((NC, N, 128), f32),
        mesh=mesh,
        scratch_types=[
            pltpu.VMEM((GRP,), i32),
            pltpu.VMEM((GRP,), i32),
            pltpu.VMEM((GRP, 128), f32),
            pltpu.VMEM((GRP, 128), f32),
            pltpu.VMEM_SHARED((N, 128), f32),
            pltpu.SemaphoreType.DMA,
        ],
    )(_sc_scatter_body)
    return k(src, dst, ew, nf)


def _merge_body(part_ref, out_ref):
    out_ref[...] = jnp.maximum(part_ref[0] + part_ref[1], 0.0)


def _merge_relu(part):
    return pl.pallas_call(
        _merge_body,
        out_shape=jax.ShapeDtypeStruct((N, 128), f32),
    )(part)


def kernel(x, cond, W_ef, b_ef, Wg_ef, Wb_ef, W_ew, b_ew, W_ep, b_ep,
           W_nf, b_nf, Wg_nf, Wb_nf, edge_index):
    src = edge_index[0].astype(i32)
    dst = edge_index[1].astype(i32)

    gb, nf = _node_precompute(x, cond, Wg_ef, Wb_ef, W_nf, b_nf, Wg_nf, Wb_nf)
    p, gbe = _sc_gather(src, dst, x, gb)
    ef, ex = _edge_mlp1(p, gbe, W_ef, b_ef, W_ew, b_ew)
    fidx = (dst[:, None] * 8 + jnp.arange(8, dtype=i32)).reshape(FLAT)
    part = _sc_seg8(fidx, ex.reshape(FLAT))
    den = _reduce_partials(part)
    ddr = _sc_dd(dst, den)
    ew = _edge_mlp2(ef, ex, ddr, dst, W_ep, b_ep)
    part2 = _sc_scatter(src, dst, ew, nf)
    return _merge_relu(part2)


# speedup vs baseline: 2.2276x; 1.1206x over previous
import functools

import jax
import jax.numpy as jnp
from jax import lax
from jax.experimental import pallas as pl
from jax.experimental.pallas import tpu as pltpu
from jax.experimental.pallas import tpu_sc as plsc

N = 10000
E = 320000
GRP = 128
NGRP = E // GRP
NC, NS = 2, 16
NW = NC * NS
NZG = N // GRP
NZT = N - NZG * GRP

f32 = jnp.float32
i32 = jnp.int32


def _node_body(x_ref, cond_ref, wgef_ref, wbef_ref, wnf_ref, bnf_ref,
               wgnf_ref, wbnf_ref, gb_ref, nf_ref):
    x = x_ref[...]
    cond = cond_ref[...]
    g = jnp.dot(cond, wgef_ref[...], preferred_element_type=f32)
    b = jnp.dot(cond, wbef_ref[...], preferred_element_type=f32)
    gb_ref[...] = jnp.concatenate([g, b], axis=1)
    hn = jnp.dot(x, wnf_ref[...], preferred_element_type=f32) + bnf_ref[...]
    nf_ref[...] = (jnp.dot(cond, wgnf_ref[...], preferred_element_type=f32) * hn
                   + jnp.dot(cond, wbnf_ref[...], preferred_element_type=f32))


def _node_precompute(x, cond, Wg_ef, Wb_ef, W_nf, b_nf, Wg_nf, Wb_nf):
    return pl.pallas_call(
        _node_body,
        out_shape=(jax.ShapeDtypeStruct((N, 128), f32),
                   jax.ShapeDtypeStruct((N, 128), f32)),
    )(x, cond, Wg_ef, Wb_ef, W_nf, b_nf.reshape(1, 128), Wg_nf, Wb_nf)


def _sc_gather_body(src_hbm, dst_hbm, x_hbm, gb_hbm, p_out, gbe_out,
                    src_v, dst_v, xs_v, xd_v, sem):
    wid = lax.axis_index("s") * NC + lax.axis_index("c")
    n_j = (NGRP - wid + NW - 1) // NW

    def grp(j, carry):
        g = wid + NW * j
        base = g * GRP
        pltpu.sync_copy(src_hbm.at[pl.ds(base, GRP)], src_v)
        pltpu.sync_copy(dst_hbm.at[pl.ds(base, GRP)], dst_v)
        pltpu.async_copy(x_hbm.at[src_v], xs_v, sem).wait()
        pltpu.async_copy(x_hbm.at[dst_v], xd_v, sem).wait()

        def mulrow(i, c):
            for k in range(8):
                s = pl.ds(k * 16, 16)
                xs_v[i, s] = xs_v[i, s] * xd_v[i, s]
            return c
        lax.fori_loop(0, GRP, mulrow, 0)
        pltpu.sync_copy(xs_v, p_out.at[pl.ds(base, GRP), :])
        pltpu.async_copy(gb_hbm.at[dst_v], xd_v, sem).wait()
        pltpu.sync_copy(xd_v, gbe_out.at[pl.ds(base, GRP), :])
        return carry
    lax.fori_loop(0, n_j, grp, 0)


def _sc_gather(src, dst, x, gb):
    mesh = plsc.VectorSubcoreMesh(core_axis_name="c", subcore_axis_name="s")
    k = functools.partial(
        pl.kernel,
        out_type=(jax.ShapeDtypeStruct((E, 128), f32),
                  jax.ShapeDtypeStruct((E, 128), f32)),
        mesh=mesh,
        scratch_types=[
            pltpu.VMEM((GRP,), i32),
            pltpu.VMEM((GRP,), i32),
            pltpu.VMEM((GRP, 128), f32),
            pltpu.VMEM((GRP, 128), f32),
            pltpu.SemaphoreType.DMA,
        ],
    )(_sc_gather_body)
    return k(src, dst, x, gb)


BE = 3200


def _edge1_body(p_ref, gb_ref, wef_ref, bef_ref, wew_ref, bew_ref,
                ef_ref, ex_ref):
    p = p_ref[...]
    gb = gb_ref[...]
    h = jnp.dot(p, wef_ref[...], preferred_element_type=f32) + bef_ref[...]
    ef = gb[:, :64] * h + gb[:, 64:]
    ef_ref[...] = ef
    logits = jnp.dot(ef, wew_ref[...], preferred_element_type=f32) + bew_ref[...]
    ex_ref[...] = jnp.exp(logits).T


def _edge_mlp1(p, gbe, W_ef, b_ef, W_ew, b_ew):
    nblk = E // BE
    return pl.pallas_call(
        _edge1_body,
        grid=(nblk,),
        in_specs=[
            pl.BlockSpec((BE, 128), lambda i: (i, 0)),
            pl.BlockSpec((BE, 128), lambda i: (i, 0)),
            pl.BlockSpec((128, 64), lambda i: (0, 0)),
            pl.BlockSpec((1, 64), lambda i: (0, 0)),
            pl.BlockSpec((64, 8), lambda i: (0, 0)),
            pl.BlockSpec((1, 8), lambda i: (0, 0)),
        ],
        out_specs=[
            pl.BlockSpec((BE, 64), lambda i: (i, 0)),
            pl.BlockSpec((8, BE), lambda i: (0, i)),
        ],
        out_shape=(jax.ShapeDtypeStruct((E, 64), f32),
                   jax.ShapeDtypeStruct((8, E), f32)),
    )(p, gbe, W_ef, b_ef.reshape(1, 64), W_ew, b_ew.reshape(1, 8))


NFLAT = N * 8


def _sc_seg8_body(dst_hbm, ext_hbm, part_out, dst_v, ext_v, acc_v, sem):
    wid = lax.axis_index("s") * NC + lax.axis_index("c")

    def zrow(i, c):
        acc_v[pl.ds(i * 16, 16)] = jnp.zeros((16,), f32)
        return c
    lax.fori_loop(0, NFLAT // 16, zrow, 0)

    n_j = (NGRP - wid + NW - 1) // NW

    def grp(j, carry):
        g = wid + NW * j
        base = g * GRP
        pltpu.sync_copy(dst_hbm.at[pl.ds(base, GRP)], dst_v)
        pltpu.sync_copy(ext_hbm.at[:, pl.ds(base, GRP)], ext_v)
        for k in range(GRP // 16):
            sl = pl.ds(k * 16, 16)
            d8 = lax.shift_left(dst_v[sl], 3)
            for h in range(8):
                plsc.addupdate_scatter(acc_v, [d8 + h], ext_v[h, sl])
        return carry
    lax.fori_loop(0, n_j, grp, 0)
    pltpu.sync_copy(acc_v, part_out.at[wid])


def _sc_seg8(dst, ext):
    mesh = plsc.VectorSubcoreMesh(core_axis_name="c", subcore_axis_name="s")
    k = functools.partial(
        pl.kernel,
        out_type=jax.ShapeDtypeStruct((NW, NFLAT), f32),
        mesh=mesh,
        scratch_types=[
            pltpu.VMEM((GRP,), i32),
            pltpu.VMEM((8, GRP), f32),
            pltpu.VMEM((NFLAT,), f32),
            pltpu.SemaphoreType.DMA,
        ],
        compiler_params=pltpu.CompilerParams(needs_layout_passes=False),
    )(_sc_seg8_body)
    return k(dst, ext)


def _reduce_body(part_ref, den_ref):
    @pl.when(pl.program_id(0) == 0)
    def _init():
        den_ref[...] = jnp.zeros_like(den_ref)
    den_ref[...] += jnp.sum(part_ref[...], axis=0)


def _reduce_partials(part):
    return pl.pallas_call(
        _reduce_body,
        grid=(NW // 8,),
        in_specs=[pl.BlockSpec((8, NFLAT), lambda i: (i, 0))],
        out_specs=pl.BlockSpec((NFLAT,), lambda i: (0,)),
        out_shape=jax.ShapeDtypeStruct((NFLAT,), f32),
    )(part)


def _sc_dd_body(dst_hbm, den_hbm, dd_out, dst_v, ridx_v, dd_v, sem):
    wid = lax.axis_index("s") * NC + lax.axis_index("c")
    n_j = (NGRP - wid + NW - 1) // NW

    def grp(j, carry):
        g = wid + NW * j
        base = g * GRP
        pltpu.sync_copy(dst_hbm.at[pl.ds(base, GRP)], dst_v)

        def shiftrow(i, c):
            sl = pl.ds(i * 16, 16)
            ridx_v[sl] = lax.shift_right_logical(dst_v[sl], 4)
            return c
        lax.fori_loop(0, GRP // 16, shiftrow, 0)
        pltpu.async_copy(den_hbm.at[ridx_v], dd_v, sem).wait()
        pltpu.sync_copy(dd_v, dd_out.at[pl.ds(base, GRP), :])
        return carry
    lax.fori_loop(0, n_j, grp, 0)


def _sc_dd(dst, den):
    mesh = plsc.VectorSubcoreMesh(core_axis_name="c", subcore_axis_name="s")
    k = functools.partial(
        pl.kernel,
        out_type=jax.ShapeDtypeStruct((E, 128), f32),
        mesh=mesh,
        scratch_types=[
            pltpu.VMEM((GRP,), i32),
            pltpu.VMEM((GRP,), i32),
            pltpu.VMEM((GRP, 128), f32),
            pltpu.SemaphoreType.DMA,
        ],
    )(_sc_dd_body)
    return k(dst, den.reshape(N // 16, 128))


def _edge2_body(ef_ref, ex_ref, ddr_ref, dst_ref, wep_ref, bep_ref, ew_ref):
    ddr = ddr_ref[...]
    mod = lax.bitwise_and(dst_ref[...], 15)
    dd = ddr[:, 0:8]
    for m in range(1, 16):
        dd = jnp.where(mod == m, ddr[:, 8 * m:8 * m + 8], dd)
    q = ex_ref[...].T / (dd + 1e-9)
    s = jnp.mean(q, axis=1, keepdims=True)
    ef2 = ef_ref[...] * s
    ep = jnp.dot(ef2, wep_ref[...], preferred_element_type=f32) + bep_ref[...]
    ew_ref[...] = s * ep


def _edge_mlp2(ef, ex, ddr, dst, W_ep, b_ep):
    nblk = E // BE
    return pl.pallas_call(
        _edge2_body,
        grid=(nblk,),
        in_specs=[
            pl.BlockSpec((BE, 64), lambda i: (i, 0)),
            pl.BlockSpec((8, BE), lambda i: (0, i)),
            pl.BlockSpec((BE, 128), lambda i: (i, 0)),
            pl.BlockSpec((BE, 1), lambda i: (i, 0)),
            pl.BlockSpec((64, 128), lambda i: (0, 0)),
            pl.BlockSpec((1, 128), lambda i: (0, 0)),
        ],
        out_specs=pl.BlockSpec((BE, 128), lambda i: (i, 0)),
        out_shape=jax.ShapeDtypeStruct((E, 128), f32),
    )(ef, ex, ddr, dst.reshape(E, 1), W_ep, b_ep.reshape(1, 128))


def _sc_scatter_body(src_hbm, dst_hbm, ew_hbm, nf_hbm, part_out,
                     src_v, dst_v, ew_v, nf_v, out_sp, sem):
    c = lax.axis_index("c")
    s = lax.axis_index("s")

    def zrow(i, cc):
        for k in range(8):
            ew_v[i, pl.ds(k * 16, 16)] = jnp.zeros((16,), f32)
        return cc
    lax.fori_loop(0, GRP, zrow, 0)

    def zcopy(j, cc):
        r = (s + NS * j) * GRP
        pltpu.sync_copy(ew_v, out_sp.at[pl.ds(r, GRP), :])
        return cc
    lax.fori_loop(0, (NZG - s + NS - 1) // NS, zcopy, 0)

    @pl.when(s == NS - 1)
    def _ztail():
        pltpu.sync_copy(ew_v.at[pl.ds(0, NZT), :],
                        out_sp.at[pl.ds(NZG * GRP, NZT), :])
    plsc.subcore_barrier()

    half = NGRP // NC
    n_j = (half - s + NS - 1) // NS

    def grp(j, carry):
        g = c * half + s + NS * j
        base = g * GRP
        pltpu.sync_copy(src_hbm.at[pl.ds(base, GRP)], src_v)
        pltpu.sync_copy(dst_hbm.at[pl.ds(base, GRP)], dst_v)
        pltpu.async_copy(nf_hbm.at[src_v], nf_v, sem).wait()
        pltpu.sync_copy(ew_hbm.at[pl.ds(base, GRP), :], ew_v)

        def mulrow(i, cc):
            for k in range(8):
                sl = pl.ds(k * 16, 16)
                ew_v[i, sl] = ew_v[i, sl] * nf_v[i, sl]
            return cc
        lax.fori_loop(0, GRP, mulrow, 0)
        pltpu.sync_copy(ew_v, out_sp.at[dst_v], add=True)
        return carry
    lax.fori_loop(0, n_j, grp, 0)
    plsc.subcore_barrier()

    def wcopy(j, cc):
        r = (s + NS * j) * GRP
        pltpu.sync_copy(out_sp.at[pl.ds(r, GRP), :], ew_v)
        pltpu.sync_copy(ew_v, part_out.at[c, pl.ds(r, GRP), :])
        return cc
    lax.fori_loop(0, (NZG - s + NS - 1) // NS, wcopy, 0)

    @pl.when(s == NS - 1)
    def _wtail():
        pltpu.sync_copy(out_sp.at[pl.ds(NZG * GRP, NZT), :],
                        ew_v.at[pl.ds(0, NZT), :])
        pltpu.sync_copy(ew_v.at[pl.ds(0, NZT), :],
                        part_out.at[c, pl.ds(NZG * GRP, NZT), :])


def _sc_scatter(src, dst, ew, nf):
    mesh = plsc.VectorSubcoreMesh(core_axis_name="c", subcore_axis_name="s")
    k = functools.partial(
        pl.kernel,
        out_type=jax.ShapeDtypeStruct((NC, N, 128), f32),
        mesh=mesh,
        scratch_types=[
            pltpu.VMEM((GRP,), i32),
            pltpu.VMEM((GRP,), i32),
            pltpu.VMEM((GRP, 128), f32),
            pltpu.VMEM((GRP, 128), f32),
            pltpu.VMEM_SHARED((N, 128), f32),
            pltpu.SemaphoreType.DMA,
        ],
    )(_sc_scatter_body)
    return k(src, dst, ew, nf)


def _merge_body(part_ref, out_ref):
    out_ref[...] = jnp.maximum(part_ref[0] + part_ref[1], 0.0)


def _merge_relu(part):
    return pl.pallas_call(
        _merge_body,
        out_shape=jax.ShapeDtypeStruct((N, 128), f32),
    )(part)


def kernel(x, cond, W_ef, b_ef, Wg_ef, Wb_ef, W_ew, b_ew, W_ep, b_ep,
           W_nf, b_nf, Wg_nf, Wb_nf, edge_index):
    src = edge_index[0].astype(i32)
    dst = edge_index[1].astype(i32)

    gb, nf = _node_precompute(x, cond, Wg_ef, Wb_ef, W_nf, b_nf, Wg_nf, Wb_nf)
    p, gbe = _sc_gather(src, dst, x, gb)
    ef, ext = _edge_mlp1(p, gbe, W_ef, b_ef, W_ew, b_ew)
    part = _sc_seg8(dst, ext)
    den = _reduce_partials(part)
    ddr = _sc_dd(dst, den)
    ew = _edge_mlp2(ef, ext, ddr, dst, W_ep, b_ep)
    part2 = _sc_scatter(src, dst, ew, nf)
    return _merge_relu(part2)


# speedup vs baseline: 2.4561x; 1.1026x over previous
import functools

import jax
import jax.numpy as jnp
from jax import lax
from jax.experimental import pallas as pl
from jax.experimental.pallas import tpu as pltpu
from jax.experimental.pallas import tpu_sc as plsc

N = 10000
E = 320000
GRP = 128
NGRP = E // GRP
NC, NS = 2, 16
NW = NC * NS
NZG = N // GRP
NZT = N - NZG * GRP

f32 = jnp.float32
i32 = jnp.int32


def _node_body(x_ref, cond_ref, wgef_ref, wbef_ref, wnf_ref, bnf_ref,
               wgnf_ref, wbnf_ref, gb_ref, nf_ref):
    x = x_ref[...]
    cond = cond_ref[...]
    g = jnp.dot(cond, wgef_ref[...], preferred_element_type=f32)
    b = jnp.dot(cond, wbef_ref[...], preferred_element_type=f32)
    gb_ref[...] = jnp.concatenate([g, b], axis=1)
    hn = jnp.dot(x, wnf_ref[...], preferred_element_type=f32) + bnf_ref[...]
    nf_ref[...] = (jnp.dot(cond, wgnf_ref[...], preferred_element_type=f32) * hn
                   + jnp.dot(cond, wbnf_ref[...], preferred_element_type=f32))


def _node_precompute(x, cond, Wg_ef, Wb_ef, W_nf, b_nf, Wg_nf, Wb_nf):
    return pl.pallas_call(
        _node_body,
        out_shape=(jax.ShapeDtypeStruct((N, 128), f32),
                   jax.ShapeDtypeStruct((N, 128), f32)),
    )(x, cond, Wg_ef, Wb_ef, W_nf, b_nf.reshape(1, 128), Wg_nf, Wb_nf)


def _sc_gather_body(src_hbm, dst_hbm, x_hbm, gb_hbm, p_out, gbe_out,
                    src_v, dst_v, xs_v, xd_v, gb_v,
                    semi, semj, sema, semb, semc, semw1, semw2):
    wid = lax.axis_index("s") * NC + lax.axis_index("c")
    n_j = (NGRP - wid + NW - 1) // NW

    def grp(j, carry):
        g = wid + NW * j
        base = g * GRP
        ci = pltpu.async_copy(src_hbm.at[pl.ds(base, GRP)], src_v, semi)
        cj = pltpu.async_copy(dst_hbm.at[pl.ds(base, GRP)], dst_v, semj)
        ci.wait()
        ca = pltpu.async_copy(x_hbm.at[src_v], xs_v, sema)
        cj.wait()
        cb = pltpu.async_copy(x_hbm.at[dst_v], xd_v, semb)
        cc = pltpu.async_copy(gb_hbm.at[dst_v], gb_v, semc)
        ca.wait()
        cb.wait()

        def mulrow(i, c):
            for k in range(8):
                s = pl.ds(k * 16, 16)
                xs_v[i, s] = xs_v[i, s] * xd_v[i, s]
            return c
        lax.fori_loop(0, GRP, mulrow, 0)
        w1 = pltpu.async_copy(xs_v, p_out.at[pl.ds(base, GRP), :], semw1)
        cc.wait()
        w2 = pltpu.async_copy(gb_v, gbe_out.at[pl.ds(base, GRP), :], semw2)
        w1.wait()
        w2.wait()
        return carry
    lax.fori_loop(0, n_j, grp, 0)


def _sc_gather(src, dst, x, gb):
    mesh = plsc.VectorSubcoreMesh(core_axis_name="c", subcore_axis_name="s")
    k = functools.partial(
        pl.kernel,
        out_type=(jax.ShapeDtypeStruct((E, 128), f32),
                  jax.ShapeDtypeStruct((E, 128), f32)),
        mesh=mesh,
        scratch_types=[
            pltpu.VMEM((GRP,), i32),
            pltpu.VMEM((GRP,), i32),
            pltpu.VMEM((GRP, 128), f32),
            pltpu.VMEM((GRP, 128), f32),
            pltpu.VMEM((GRP, 128), f32),
        ] + [pltpu.SemaphoreType.DMA] * 7,
    )(_sc_gather_body)
    return k(src, dst, x, gb)


BE = 3200


def _edge1_body(p_ref, gb_ref, wef_ref, bef_ref, wew_ref, bew_ref,
                ef_ref, ex_ref):
    p = p_ref[...]
    gb = gb_ref[...]
    h = jnp.dot(p, wef_ref[...], preferred_element_type=f32) + bef_ref[...]
    ef = gb[:, :64] * h + gb[:, 64:]
    ef_ref[...] = ef
    logits = jnp.dot(ef, wew_ref[...], preferred_element_type=f32) + bew_ref[...]
    ex_ref[...] = jnp.exp(logits).T


def _edge_mlp1(p, gbe, W_ef, b_ef, W_ew, b_ew):
    nblk = E // BE
    return pl.pallas_call(
        _edge1_body,
        grid=(nblk,),
        in_specs=[
            pl.BlockSpec((BE, 128), lambda i: (i, 0)),
            pl.BlockSpec((BE, 128), lambda i: (i, 0)),
            pl.BlockSpec((128, 64), lambda i: (0, 0)),
            pl.BlockSpec((1, 64), lambda i: (0, 0)),
            pl.BlockSpec((64, 8), lambda i: (0, 0)),
            pl.BlockSpec((1, 8), lambda i: (0, 0)),
        ],
        out_specs=[
            pl.BlockSpec((BE, 64), lambda i: (i, 0)),
            pl.BlockSpec((8, BE), lambda i: (0, i)),
        ],
        out_shape=(jax.ShapeDtypeStruct((E, 64), f32),
                   jax.ShapeDtypeStruct((8, E), f32)),
    )(p, gbe, W_ef, b_ef.reshape(1, 64), W_ew, b_ew.reshape(1, 8))


NFLAT = N * 8


def _sc_seg8_body(dst_hbm, ext_hbm, part_out, dst_v, ext_v, acc_v, sem):
    wid = lax.axis_index("s") * NC + lax.axis_index("c")

    def zrow(i, c):
        acc_v[pl.ds(i * 16, 16)] = jnp.zeros((16,), f32)
        return c
    lax.fori_loop(0, NFLAT // 16, zrow, 0)

    n_j = (NGRP - wid + NW - 1) // NW

    def grp(j, carry):
        g = wid + NW * j
        base = g * GRP
        pltpu.sync_copy(dst_hbm.at[pl.ds(base, GRP)], dst_v)
        pltpu.sync_copy(ext_hbm.at[:, pl.ds(base, GRP)], ext_v)
        for k in range(GRP // 16):
            sl = pl.ds(k * 16, 16)
            d8 = lax.shift_left(dst_v[sl], 3)
            for h in range(8):
                plsc.addupdate_scatter(acc_v, [d8 + h], ext_v[h, sl])
        return carry
    lax.fori_loop(0, n_j, grp, 0)
    pltpu.sync_copy(acc_v, part_out.at[wid])


def _sc_seg8(dst, ext):
    mesh = plsc.VectorSubcoreMesh(core_axis_name="c", subcore_axis_name="s")
    k = functools.partial(
        pl.kernel,
        out_type=jax.ShapeDtypeStruct((NW, NFLAT), f32),
        mesh=mesh,
        scratch_types=[
            pltpu.VMEM((GRP,), i32),
            pltpu.VMEM((8, GRP), f32),
            pltpu.VMEM((NFLAT,), f32),
            pltpu.SemaphoreType.DMA,
        ],
        compiler_params=pltpu.CompilerParams(needs_layout_passes=False),
    )(_sc_seg8_body)
    return k(dst, ext)


def _reduce_body(part_ref, den_ref):
    @pl.when(pl.program_id(0) == 0)
    def _init():
        den_ref[...] = jnp.zeros_like(den_ref)
    den_ref[...] += jnp.sum(part_ref[...], axis=0)


def _reduce_partials(part):
    return pl.pallas_call(
        _reduce_body,
        grid=(NW // 8,),
        in_specs=[pl.BlockSpec((8, NFLAT), lambda i: (i, 0))],
        out_specs=pl.BlockSpec((NFLAT,), lambda i: (0,)),
        out_shape=jax.ShapeDtypeStruct((NFLAT,), f32),
    )(part)


def _sc_dd_body(dst_hbm, den_hbm, dd_out, dst_v, ridx_v, dd_v, sem):
    wid = lax.axis_index("s") * NC + lax.axis_index("c")
    n_j = (NGRP - wid + NW - 1) // NW

    def grp(j, carry):
        g = wid + NW * j
        base = g * GRP
        pltpu.sync_copy(dst_hbm.at[pl.ds(base, GRP)], dst_v)

        def shiftrow(i, c):
            sl = pl.ds(i * 16, 16)
            ridx_v[sl] = lax.shift_right_logical(dst_v[sl], 4)
            return c
        lax.fori_loop(0, GRP // 16, shiftrow, 0)
        pltpu.async_copy(den_hbm.at[ridx_v], dd_v, sem).wait()
        pltpu.sync_copy(dd_v, dd_out.at[pl.ds(base, GRP), :])
        return carry
    lax.fori_loop(0, n_j, grp, 0)


def _sc_dd(dst, den):
    mesh = plsc.VectorSubcoreMesh(core_axis_name="c", subcore_axis_name="s")
    k = functools.partial(
        pl.kernel,
        out_type=jax.ShapeDtypeStruct((E, 128), f32),
        mesh=mesh,
        scratch_types=[
            pltpu.VMEM((GRP,), i32),
            pltpu.VMEM((GRP,), i32),
            pltpu.VMEM((GRP, 128), f32),
            pltpu.SemaphoreType.DMA,
        ],
    )(_sc_dd_body)
    return k(dst, den.reshape(N // 16, 128))


def _edge2_body(ef_ref, ex_ref, ddr_ref, dst_ref, wep_ref, bep_ref, ew_ref):
    ddr = ddr_ref[...]
    mod = lax.bitwise_and(dst_ref[...], 15)
    dd = ddr[:, 0:8]
    for m in range(1, 16):
        dd = jnp.where(mod == m, ddr[:, 8 * m:8 * m + 8], dd)
    q = ex_ref[...].T / (dd + 1e-9)
    s = jnp.mean(q, axis=1, keepdims=True)
    ef2 = ef_ref[...] * s
    ep = jnp.dot(ef2, wep_ref[...], preferred_element_type=f32) + bep_ref[...]
    ew_ref[...] = s * ep


def _edge_mlp2(ef, ex, ddr, dst, W_ep, b_ep):
    nblk = E // BE
    return pl.pallas_call(
        _edge2_body,
        grid=(nblk,),
        in_specs=[
            pl.BlockSpec((BE, 64), lambda i: (i, 0)),
            pl.BlockSpec((8, BE), lambda i: (0, i)),
            pl.BlockSpec((BE, 128), lambda i: (i, 0)),
            pl.BlockSpec((BE, 1), lambda i: (i, 0)),
            pl.BlockSpec((64, 128), lambda i: (0, 0)),
            pl.BlockSpec((1, 128), lambda i: (0, 0)),
        ],
        out_specs=pl.BlockSpec((BE, 128), lambda i: (i, 0)),
        out_shape=jax.ShapeDtypeStruct((E, 128), f32),
    )(ef, ex, ddr, dst.reshape(E, 1), W_ep, b_ep.reshape(1, 128))


def _sc_scatter_body(src_hbm, dst_hbm, ew_hbm, nf_hbm, part_out,
                     src_v, dst_v, ew_v, nf_v, out_sp,
                     sem, semi, semj, semb):
    c = lax.axis_index("c")
    s = lax.axis_index("s")

    def zrow(i, cc):
        for k in range(8):
            ew_v[i, pl.ds(k * 16, 16)] = jnp.zeros((16,), f32)
        return cc
    lax.fori_loop(0, GRP, zrow, 0)

    def zcopy(j, cc):
        r = (s + NS * j) * GRP
        pltpu.sync_copy(ew_v, out_sp.at[pl.ds(r, GRP), :])
        return cc
    lax.fori_loop(0, (NZG - s + NS - 1) // NS, zcopy, 0)

    @pl.when(s == NS - 1)
    def _ztail():
        pltpu.sync_copy(ew_v.at[pl.ds(0, NZT), :],
                        out_sp.at[pl.ds(NZG * GRP, NZT), :])
    plsc.subcore_barrier()

    half = NGRP // NC
    n_j = (half - s + NS - 1) // NS

    def grp(j, carry):
        g = c * half + s + NS * j
        base = g * GRP
        ci = pltpu.async_copy(src_hbm.at[pl.ds(base, GRP)], src_v, semi)
        cj = pltpu.async_copy(dst_hbm.at[pl.ds(base, GRP)], dst_v, semj)
        cb = pltpu.async_copy(ew_hbm.at[pl.ds(base, GRP), :], ew_v, semb)
        ci.wait()
        ca = pltpu.async_copy(nf_hbm.at[src_v], nf_v, sem)
        ca.wait()
        cb.wait()

        def mulrow(i, cc):
            for k in range(8):
                sl = pl.ds(k * 16, 16)
                ew_v[i, sl] = ew_v[i, sl] * nf_v[i, sl]
            return cc
        lax.fori_loop(0, GRP, mulrow, 0)
        cj.wait()
        pltpu.sync_copy(ew_v, out_sp.at[dst_v], add=True)
        return carry
    lax.fori_loop(0, n_j, grp, 0)
    plsc.subcore_barrier()

    def wcopy(j, cc):
        r = (s + NS * j) * GRP
        pltpu.sync_copy(out_sp.at[pl.ds(r, GRP), :], ew_v)
        pltpu.sync_copy(ew_v, part_out.at[c, pl.ds(r, GRP), :])
        return cc
    lax.fori_loop(0, (NZG - s + NS - 1) // NS, wcopy, 0)

    @pl.when(s == NS - 1)
    def _wtail():
        pltpu.sync_copy(out_sp.at[pl.ds(NZG * GRP, NZT), :],
                        ew_v.at[pl.ds(0, NZT), :])
        pltpu.sync_copy(ew_v.at[pl.ds(0, NZT), :],
                        part_out.at[c, pl.ds(NZG * GRP, NZT), :])


def _sc_scatter(src, dst, ew, nf):
    mesh = plsc.VectorSubcoreMesh(core_axis_name="c", subcore_axis_name="s")
    k = functools.partial(
        pl.kernel,
        out_type=jax.ShapeDtypeStruct((NC, N, 128), f32),
        mesh=mesh,
        scratch_types=[
            pltpu.VMEM((GRP,), i32),
            pltpu.VMEM((GRP,), i32),
            pltpu.VMEM((GRP, 128), f32),
            pltpu.VMEM((GRP, 128), f32),
            pltpu.VMEM_SHARED((N, 128), f32),
        ] + [pltpu.SemaphoreType.DMA] * 4,
    )(_sc_scatter_body)
    return k(src, dst, ew, nf)


def _merge_body(part_ref, out_ref):
    out_ref[...] = jnp.maximum(part_ref[0] + part_ref[1], 0.0)


def _merge_relu(part):
    return pl.pallas_call(
        _merge_body,
        out_shape=jax.ShapeDtypeStruct((N, 128), f32),
    )(part)


def kernel(x, cond, W_ef, b_ef, Wg_ef, Wb_ef, W_ew, b_ew, W_ep, b_ep,
           W_nf, b_nf, Wg_nf, Wb_nf, edge_index):
    src = edge_index[0].astype(i32)
    dst = edge_index[1].astype(i32)

    gb, nf = _node_precompute(x, cond, Wg_ef, Wb_ef, W_nf, b_nf, Wg_nf, Wb_nf)
    p, gbe = _sc_gather(src, dst, x, gb)
    ef, ext = _edge_mlp1(p, gbe, W_ef, b_ef, W_ew, b_ew)
    part = _sc_seg8(dst, ext)
    den = _reduce_partials(part)
    ddr = _sc_dd(dst, den)
    ew = _edge_mlp2(ef, ext, ddr, dst, W_ep, b_ep)
    part2 = _sc_scatter(src, dst, ew, nf)
    return _merge_relu(part2)


# speedup vs baseline: 4.1627x; 1.6948x over previous
import functools

import jax
import jax.numpy as jnp
from jax import lax
from jax.experimental import pallas as pl
from jax.experimental.pallas import tpu as pltpu
from jax.experimental.pallas import tpu_sc as plsc

N = 10000
E = 320000
GRP = 128
NGRP = E // GRP
NC, NS = 2, 16
NW = NC * NS
NZG = N // GRP
NZT = N - NZG * GRP

f32 = jnp.float32
i32 = jnp.int32


def _node_body(x_ref, cond_ref, wgef_ref, wbef_ref, wnf_ref, bnf_ref,
               wgnf_ref, wbnf_ref, gb_ref, nf_ref):
    x = x_ref[...]
    cond = cond_ref[...]
    g = jnp.dot(cond, wgef_ref[...], preferred_element_type=f32)
    b = jnp.dot(cond, wbef_ref[...], preferred_element_type=f32)
    gb_ref[...] = jnp.concatenate([g, b], axis=1)
    hn = jnp.dot(x, wnf_ref[...], preferred_element_type=f32) + bnf_ref[...]
    nf_ref[...] = (jnp.dot(cond, wgnf_ref[...], preferred_element_type=f32) * hn
                   + jnp.dot(cond, wbnf_ref[...], preferred_element_type=f32))


def _node_precompute(x, cond, Wg_ef, Wb_ef, W_nf, b_nf, Wg_nf, Wb_nf):
    return pl.pallas_call(
        _node_body,
        out_shape=(jax.ShapeDtypeStruct((N, 128), f32),
                   jax.ShapeDtypeStruct((N, 128), f32)),
    )(x, cond, Wg_ef, Wb_ef, W_nf, b_nf.reshape(1, 128), Wg_nf, Wb_nf)


def _sc_gather_body(src_hbm, dst_hbm, x_hbm, gb_hbm, p_out, gbe_out,
                    src_v, dst_v, xs_v, xd_v, gb_v,
                    semi, semj, sema, semb, semc, semw1, semw2):
    wid = lax.axis_index("s") * NC + lax.axis_index("c")
    n_j = (NGRP - wid + NW - 1) // NW

    def grp(j, carry):
        g = wid + NW * j
        base = g * GRP
        ci = pltpu.async_copy(src_hbm.at[pl.ds(base, GRP)], src_v, semi)
        cj = pltpu.async_copy(dst_hbm.at[pl.ds(base, GRP)], dst_v, semj)
        ci.wait()
        ca = pltpu.async_copy(x_hbm.at[src_v], xs_v, sema)
        cj.wait()
        cb = pltpu.async_copy(x_hbm.at[dst_v], xd_v, semb)
        cc = pltpu.async_copy(gb_hbm.at[dst_v], gb_v, semc)
        ca.wait()
        cb.wait()

        def mulrow(i, c):
            for k in range(8):
                s = pl.ds(k * 16, 16)
                xs_v[i, s] = xs_v[i, s] * xd_v[i, s]
            return c
        lax.fori_loop(0, GRP, mulrow, 0)
        w1 = pltpu.async_copy(xs_v, p_out.at[pl.ds(base, GRP), :], semw1)
        cc.wait()
        w2 = pltpu.async_copy(gb_v, gbe_out.at[pl.ds(base, GRP), :], semw2)
        w1.wait()
        w2.wait()
        return carry
    lax.fori_loop(0, n_j, grp, 0)


def _sc_gather(src, dst, x, gb):
    mesh = plsc.VectorSubcoreMesh(core_axis_name="c", subcore_axis_name="s")
    k = functools.partial(
        pl.kernel,
        out_type=(jax.ShapeDtypeStruct((E, 128), f32),
                  jax.ShapeDtypeStruct((E, 128), f32)),
        mesh=mesh,
        scratch_types=[
            pltpu.VMEM((GRP,), i32),
            pltpu.VMEM((GRP,), i32),
            pltpu.VMEM((GRP, 128), f32),
            pltpu.VMEM((GRP, 128), f32),
            pltpu.VMEM((GRP, 128), f32),
        ] + [pltpu.SemaphoreType.DMA] * 7,
    )(_sc_gather_body)
    return k(src, dst, x, gb)


BE = 3200


def _edge1_body(p_ref, gb_ref, wef_ref, bef_ref, wew_ref, bew_ref,
                ef_ref, ex_ref):
    p = p_ref[...]
    gb = gb_ref[...]
    h = jnp.dot(p, wef_ref[...], preferred_element_type=f32) + bef_ref[...]
    ef = gb[:, :64] * h + gb[:, 64:]
    ef_ref[...] = ef
    logits_t = lax.dot_general(wew_ref[...], ef, (((0,), (1,)), ((), ())),
                               preferred_element_type=f32) + bew_ref[...]
    ex_ref[...] = jnp.exp(logits_t)


def _edge_mlp1(p, gbe, W_ef, b_ef, W_ew, b_ew):
    nblk = E // BE
    return pl.pallas_call(
        _edge1_body,
        grid=(nblk,),
        in_specs=[
            pl.BlockSpec((BE, 128), lambda i: (i, 0)),
            pl.BlockSpec((BE, 128), lambda i: (i, 0)),
            pl.BlockSpec((128, 64), lambda i: (0, 0)),
            pl.BlockSpec((1, 64), lambda i: (0, 0)),
            pl.BlockSpec((64, 8), lambda i: (0, 0)),
            pl.BlockSpec((8, 1), lambda i: (0, 0)),
        ],
        out_specs=[
            pl.BlockSpec((BE, 64), lambda i: (i, 0)),
            pl.BlockSpec((8, BE), lambda i: (0, i)),
        ],
        out_shape=(jax.ShapeDtypeStruct((E, 64), f32),
                   jax.ShapeDtypeStruct((8, E), f32)),
    )(p, gbe, W_ef, b_ef.reshape(1, 64), W_ew, b_ew.reshape(8, 1))


NFLAT = N * 8


def _sc_seg8_body(dst_hbm, ext_hbm, part_out, dst_v, ext_v, acc_v, sem):
    wid = lax.axis_index("s") * NC + lax.axis_index("c")

    def zrow(i, c):
        acc_v[pl.ds(i * 16, 16)] = jnp.zeros((16,), f32)
        return c
    lax.fori_loop(0, NFLAT // 16, zrow, 0)

    n_j = (NGRP - wid + NW - 1) // NW

    def grp(j, carry):
        g = wid + NW * j
        base = g * GRP
        pltpu.sync_copy(dst_hbm.at[pl.ds(base, GRP)], dst_v)
        pltpu.sync_copy(ext_hbm.at[:, pl.ds(base, GRP)], ext_v)
        for k in range(GRP // 16):
            sl = pl.ds(k * 16, 16)
            d8 = lax.shift_left(dst_v[sl], 3)
            for h in range(8):
                plsc.addupdate_scatter(acc_v, [d8 + h], ext_v[h, sl])
        return carry
    lax.fori_loop(0, n_j, grp, 0)
    pltpu.sync_copy(acc_v, part_out.at[wid])


def _sc_seg8(dst, ext):
    mesh = plsc.VectorSubcoreMesh(core_axis_name="c", subcore_axis_name="s")
    k = functools.partial(
        pl.kernel,
        out_type=jax.ShapeDtypeStruct((NW, NFLAT), f32),
        mesh=mesh,
        scratch_types=[
            pltpu.VMEM((GRP,), i32),
            pltpu.VMEM((8, GRP), f32),
            pltpu.VMEM((NFLAT,), f32),
            pltpu.SemaphoreType.DMA,
        ],
        compiler_params=pltpu.CompilerParams(needs_layout_passes=False),
    )(_sc_seg8_body)
    return k(dst, ext)


def _reduce_body(part_ref, den_ref):
    @pl.when(pl.program_id(0) == 0)
    def _init():
        den_ref[...] = jnp.zeros_like(den_ref)
    den_ref[...] += jnp.sum(part_ref[...], axis=0)


def _reduce_partials(part):
    return pl.pallas_call(
        _reduce_body,
        grid=(NW // 8,),
        in_specs=[pl.BlockSpec((8, NFLAT), lambda i: (i, 0))],
        out_specs=pl.BlockSpec((NFLAT,), lambda i: (0,)),
        out_shape=jax.ShapeDtypeStruct((NFLAT,), f32),
    )(part)


def _sc_dd_body(dst_hbm, den_hbm, dd_out, dst_v, ridx_v, dd_v, sem):
    wid = lax.axis_index("s") * NC + lax.axis_index("c")
    n_j = (NGRP - wid + NW - 1) // NW

    def grp(j, carry):
        g = wid + NW * j
        base = g * GRP
        pltpu.sync_copy(dst_hbm.at[pl.ds(base, GRP)], dst_v)

        def shiftrow(i, c):
            sl = pl.ds(i * 16, 16)
            ridx_v[sl] = lax.shift_right_logical(dst_v[sl], 4)
            return c
        lax.fori_loop(0, GRP // 16, shiftrow, 0)
        pltpu.async_copy(den_hbm.at[ridx_v], dd_v, sem).wait()
        pltpu.sync_copy(dd_v, dd_out.at[pl.ds(base, GRP), :])
        return carry
    lax.fori_loop(0, n_j, grp, 0)


def _sc_dd(dst, den):
    mesh = plsc.VectorSubcoreMesh(core_axis_name="c", subcore_axis_name="s")
    k = functools.partial(
        pl.kernel,
        out_type=jax.ShapeDtypeStruct((E, 128), f32),
        mesh=mesh,
        scratch_types=[
            pltpu.VMEM((GRP,), i32),
            pltpu.VMEM((GRP,), i32),
            pltpu.VMEM((GRP, 128), f32),
            pltpu.SemaphoreType.DMA,
        ],
    )(_sc_dd_body)
    return k(dst, den.reshape(N // 16, 128))


def _edge2_body(ef_ref, ex_ref, ddr_ref, dst_ref, wep_ref, bep_ref, ew_ref):
    ddr = ddr_ref[...]
    colh = lax.bitwise_and(lax.broadcasted_iota(i32, (8, 128), 1), 7)
    rowh = lax.broadcasted_iota(i32, (8, 128), 0)
    t8 = (colh == rowh).astype(f32)
    exw = lax.dot_general(ex_ref[...], t8, (((0,), (0,)), ((), ())),
                          preferred_element_type=f32)
    mod = lax.bitwise_and(dst_ref[...], 15)
    lane16 = lax.shift_right_logical(
        lax.broadcasted_iota(i32, (BE, 128), 1), 3)
    qw = jnp.where(lane16 == mod, exw / (ddr + 1e-9), 0.0)
    s = jnp.sum(qw, axis=1, keepdims=True) * 0.125
    ef2 = ef_ref[...] * s
    ep = jnp.dot(ef2, wep_ref[...], preferred_element_type=f32) + bep_ref[...]
    ew_ref[...] = s * ep


def _edge_mlp2(ef, ex, ddr, dst, W_ep, b_ep):
    nblk = E // BE
    return pl.pallas_call(
        _edge2_body,
        grid=(nblk,),
        in_specs=[
            pl.BlockSpec((BE, 64), lambda i: (i, 0)),
            pl.BlockSpec((8, BE), lambda i: (0, i)),
            pl.BlockSpec((BE, 128), lambda i: (i, 0)),
            pl.BlockSpec((BE, 1), lambda i: (i, 0)),
            pl.BlockSpec((64, 128), lambda i: (0, 0)),
            pl.BlockSpec((1, 128), lambda i: (0, 0)),
        ],
        out_specs=pl.BlockSpec((BE, 128), lambda i: (i, 0)),
        out_shape=jax.ShapeDtypeStruct((E, 128), f32),
    )(ef, ex, ddr, dst.reshape(E, 1), W_ep, b_ep.reshape(1, 128))


def _sc_scatter_body(src_hbm, dst_hbm, ew_hbm, nf_hbm, part_out,
                     src_v, dst_v, ew_v, nf_v, out_sp,
                     sem, semi, semj, semb):
    c = lax.axis_index("c")
    s = lax.axis_index("s")

    def zrow(i, cc):
        for k in range(8):
            ew_v[i, pl.ds(k * 16, 16)] = jnp.zeros((16,), f32)
        return cc
    lax.fori_loop(0, GRP, zrow, 0)

    def zcopy(j, cc):
        r = (s + NS * j) * GRP
        pltpu.sync_copy(ew_v, out_sp.at[pl.ds(r, GRP), :])
        return cc
    lax.fori_loop(0, (NZG - s + NS - 1) // NS, zcopy, 0)

    @pl.when(s == NS - 1)
    def _ztail():
        pltpu.sync_copy(ew_v.at[pl.ds(0, NZT), :],
                        out_sp.at[pl.ds(NZG * GRP, NZT), :])
    plsc.subcore_barrier()

    half = NGRP // NC
    n_j = (half - s + NS - 1) // NS

    def grp(j, carry):
        g = c * half + s + NS * j
        base = g * GRP
        ci = pltpu.async_copy(src_hbm.at[pl.ds(base, GRP)], src_v, semi)
        cj = pltpu.async_copy(dst_hbm.at[pl.ds(base, GRP)], dst_v, semj)
        cb = pltpu.async_copy(ew_hbm.at[pl.ds(base, GRP), :], ew_v, semb)
        ci.wait()
        ca = pltpu.async_copy(nf_hbm.at[src_v], nf_v, sem)
        ca.wait()
        cb.wait()

        def mulrow(i, cc):
            for k in range(8):
                sl = pl.ds(k * 16, 16)
                ew_v[i, sl] = ew_v[i, sl] * nf_v[i, sl]
            return cc
        lax.fori_loop(0, GRP, mulrow, 0)
        cj.wait()
        pltpu.sync_copy(ew_v, out_sp.at[dst_v], add=True)
        return carry
    lax.fori_loop(0, n_j, grp, 0)
    plsc.subcore_barrier()

    def wcopy(j, cc):
        r = (s + NS * j) * GRP
        pltpu.sync_copy(out_sp.at[pl.ds(r, GRP), :], ew_v)
        pltpu.sync_copy(ew_v, part_out.at[c, pl.ds(r, GRP), :])
        return cc
    lax.fori_loop(0, (NZG - s + NS - 1) // NS, wcopy, 0)

    @pl.when(s == NS - 1)
    def _wtail():
        pltpu.sync_copy(out_sp.at[pl.ds(NZG * GRP, NZT), :],
                        ew_v.at[pl.ds(0, NZT), :])
        pltpu.sync_copy(ew_v.at[pl.ds(0, NZT), :],
                        part_out.at[c, pl.ds(NZG * GRP, NZT), :])


def _sc_scatter(src, dst, ew, nf):
    mesh = plsc.VectorSubcoreMesh(core_axis_name="c", subcore_axis_name="s")
    k = functools.partial(
        pl.kernel,
        out_type=jax.ShapeDtypeStruct((NC, N, 128), f32),
        mesh=mesh,
        scratch_types=[
            pltpu.VMEM((GRP,), i32),
            pltpu.VMEM((GRP,), i32),
            pltpu.VMEM((GRP, 128), f32),
            pltpu.VMEM((GRP, 128), f32),
            pltpu.VMEM_SHARED((N, 128), f32),
        ] + [pltpu.SemaphoreType.DMA] * 4,
    )(_sc_scatter_body)
    return k(src, dst, ew, nf)


def _merge_body(part_ref, out_ref):
    out_ref[...] = jnp.maximum(part_ref[0] + part_ref[1], 0.0)


def _merge_relu(part):
    return pl.pallas_call(
        _merge_body,
        out_shape=jax.ShapeDtypeStruct((N, 128), f32),
    )(part)


def kernel(x, cond, W_ef, b_ef, Wg_ef, Wb_ef, W_ew, b_ew, W_ep, b_ep,
           W_nf, b_nf, Wg_nf, Wb_nf, edge_index):
    src = edge_index[0].astype(i32)
    dst = edge_index[1].astype(i32)

    gb, nf = _node_precompute(x, cond, Wg_ef, Wb_ef, W_nf, b_nf, Wg_nf, Wb_nf)
    p, gbe = _sc_gather(src, dst, x, gb)
    ef, ext = _edge_mlp1(p, gbe, W_ef, b_ef, W_ew, b_ew)
    part = _sc_seg8(dst, ext)
    den = _reduce_partials(part)
    ddr = _sc_dd(dst, den)
    ew = _edge_mlp2(ef, ext, ddr, dst, W_ep, b_ep)
    part2 = _sc_scatter(src, dst, ew, nf)
    return _merge_relu(part2)
